# Initial kernel scaffold; baseline (speedup 1.0000x reference)
import jax
import jax.numpy as jnp
from jax.experimental import pallas as pl


def kernel(vis_feat, text_feat):
    raise NotImplementedError("write your pallas kernel here")


# speedup vs baseline: 11.4396x; 11.4396x over previous
import functools

import jax
import jax.numpy as jnp
from jax import lax
from jax.experimental import pallas as pl
from jax.experimental.pallas import tpu as pltpu

_K = 64
_MASK = 10000.0


def _monotone_u32(x):
    b = lax.bitcast_convert_type(x, jnp.uint32)
    neg = b >= jnp.uint32(0x80000000)
    return jnp.where(neg, ~b, b | jnp.uint32(0x80000000))


def _inv_monotone_u32(u):
    pos = u >= jnp.uint32(0x80000000)
    b = jnp.where(pos, u ^ jnp.uint32(0x80000000), ~u)
    return lax.bitcast_convert_type(b, jnp.float32)


def _hardneg_body(q_ref, k_ref, out_ref, *, blk, bsz, nblk):
    g = pl.program_id(0)
    b = g % nblk

    q = q_ref[...]
    k = k_ref[...]
    s = jax.lax.dot_general(
        q, k, (((1,), (1,)), ((), ())), preferred_element_type=jnp.float32
    )

    rows = b * blk + lax.broadcasted_iota(jnp.int32, (blk, bsz), 0)
    cols = lax.broadcasted_iota(jnp.int32, (blk, bsz), 1)
    is_diag = rows == cols
    diag = jnp.sum(jnp.where(is_diag, s, 0.0), axis=1, keepdims=True)
    s = s - jnp.where(is_diag, _MASK, 0.0)

    m = jnp.max(s, axis=1, keepdims=True)
    u = _monotone_u32(s)

    lo0 = jnp.zeros((blk, 1), jnp.uint32)
    hi0 = jnp.full((blk, 1), 0xFFFFFFFF, jnp.uint32)

    def body(_, carry):
        lo, hi = carry
        mid = lo + ((hi - lo) // 2) + ((hi - lo) & 1)
        cnt = jnp.sum((u >= mid).astype(jnp.int32), axis=1, keepdims=True)
        ok = cnt >= _K
        return jnp.where(ok, mid, lo), jnp.where(ok, hi, mid - 1)

    t_u, _ = lax.fori_loop(0, 32, body, (lo0, hi0))
    t_f = _inv_monotone_u32(t_u)

    keep = u >= t_u
    cnt = jnp.sum(keep.astype(jnp.float32), axis=1, keepdims=True)
    sums = jnp.sum(jnp.where(keep, jnp.exp(s - m), 0.0), axis=1, keepdims=True)
    sums = sums - (cnt - float(_K)) * jnp.exp(t_f - m)

    big = jnp.maximum(m, diag)
    lse = jnp.log(jnp.exp(diag - big) + sums * jnp.exp(m - big)) + big
    part = jnp.sum(lse - diag, keepdims=True) / float(bsz)

    @pl.when(g == 0)
    def _():
        out_ref[...] = jnp.zeros((1, 1), jnp.float32)

    out_ref[...] += part


def kernel(vis_feat, text_feat):
    bsz, dim = vis_feat.shape
    blk = 512
    nblk = bsz // blk
    q = jnp.concatenate([text_feat, vis_feat], axis=0)
    km = jnp.concatenate([vis_feat, text_feat], axis=0)

    out = pl.pallas_call(
        functools.partial(_hardneg_body, blk=blk, bsz=bsz, nblk=nblk),
        grid=(2 * nblk,),
        in_specs=[
            pl.BlockSpec((blk, dim), lambda g: (g, 0)),
            pl.BlockSpec((bsz, dim), lambda g, nblk=nblk: (g // nblk, 0)),
        ],
        out_specs=pl.BlockSpec((1, 1), lambda g: (0, 0)),
        out_shape=jax.ShapeDtypeStruct((1, 1), jnp.float32),
    )(q, km)
    return out[0, 0]
